# Initial kernel scaffold; baseline (speedup 1.0000x reference)
import numpy as np
import jax
import jax.numpy as jnp
from jax.experimental import pallas as pl
from jax.experimental.pallas import tpu as pltpu

LEN = 128


def _frame_indices(T: int) -> np.ndarray:
    f = np.linspace(0.0, T - 1, LEN, dtype=np.float32)
    return np.clip(np.rint(f), 0, T - 1).astype(np.int32)


def _copy_body(x_ref, o_ref):
    o_ref[...] = x_ref[...]


def kernel(x):
    T, C, H, W = x.shape
    idx = jnp.asarray(_frame_indices(T))

    return pl.pallas_call(
        _copy_body,
        grid=(LEN,),
        in_specs=[pl.BlockSpec((1, C, H, W), lambda i: (idx[i], 0, 0, 0))],
        out_specs=pl.BlockSpec((1, C, H, W), lambda i: (i, 0, 0, 0)),
        out_shape=jax.ShapeDtypeStruct((LEN, C, H, W), x.dtype),
        compiler_params=pltpu.CompilerParams(
            dimension_semantics=("arbitrary",),
        ),
    )(x)


# speedup vs baseline: 1.0272x; 1.0272x over previous
import numpy as np
import jax
import jax.numpy as jnp
from jax.experimental import pallas as pl
from jax.experimental.pallas import tpu as pltpu

LEN = 128


def _frame_indices(T: int) -> np.ndarray:
    f = np.linspace(0.0, T - 1, LEN, dtype=np.float32)
    return np.clip(np.rint(f), 0, T - 1).astype(np.int32)


def _copy_body(x_ref, o_ref):
    o_ref[...] = x_ref[...]


def kernel(x):
    T, C, H, W = x.shape
    a, b = 2 * (T - 1), LEN - 1

    return pl.pallas_call(
        _copy_body,
        grid=(LEN,),
        in_specs=[pl.BlockSpec((1, C, H, W), lambda i: ((i * a + b) // (2 * b), 0, 0, 0))],
        out_specs=pl.BlockSpec((1, C, H, W), lambda i: (i, 0, 0, 0)),
        out_shape=jax.ShapeDtypeStruct((LEN, C, H, W), x.dtype),
        compiler_params=pltpu.CompilerParams(
            dimension_semantics=("arbitrary",),
        ),
    )(x)
